# Initial kernel scaffold; baseline (speedup 1.0000x reference)
import jax
import jax.numpy as jnp
from jax.experimental import pallas as pl


def kernel(user_embed, item_embed, edge_weight, gumbel1, gumbel2, edge_index, users, pos_items, neg_items):
    raise NotImplementedError("write your pallas kernel here")


# speedup vs baseline: 2.0687x; 2.0687x over previous
import functools

import jax
import jax.numpy as jnp
from jax import lax
from jax.experimental import pallas as pl
from jax.experimental.pallas import tpu as pltpu
from jax.experimental.pallas import tpu_sc as plsc

N_USERS = 10000
N_ITEMS = 40000
N_NODES = 50000
DIM = 64
N_HOPS = 3
NNZ = 800000
B = 4096
K_NEG = 16
ALPHA = 0.5
DECAY = 1e-4

NC = 2
NS = 16

EROWS = 6400
ROWS_PER_TILE = EROWS // NS
CHUNK_R = 2
NCHUNK = ROWS_PER_TILE // CHUNK_R
CE = CHUNK_R * 128
HALF = N_NODES // NC
STRIPE = 1568
ACC_ROWS = HALF
LAST_WB = HALF - (NS - 1) * STRIPE

def _mesh():
    return plsc.VectorSubcoreMesh(
        core_axis_name="c", subcore_axis_name="s", num_cores=NC, num_subcores=NS
    )


def _hop_body(agg, rowp, colp, wflat, zeros, out,
              acc, colbuf, dstbuf, wbuf, rows, gsem):
    c = lax.axis_index("c")
    s = lax.axis_index("s")
    base_row = c * HALF
    tile_row0 = s * ROWS_PER_TILE

    @pl.when(s != NS - 1)
    def _():
        pltpu.sync_copy(zeros, acc.at[pl.ds(s * STRIPE, STRIPE)])

    @pl.when(s == NS - 1)
    def _():
        pltpu.sync_copy(
            zeros.at[pl.ds(0, LAST_WB)], acc.at[pl.ds(s * STRIPE, LAST_WB)]
        )

    plsc.subcore_barrier()

    def chunk_body(g, carry):
        r0 = tile_row0 + g * CHUNK_R
        pltpu.sync_copy(rowp.at[pl.ds(r0, CHUNK_R)], dstbuf)
        pltpu.sync_copy(colp.at[pl.ds(r0, CHUNK_R)], colbuf)
        pltpu.sync_copy(wflat.at[pl.ds(r0 * 128, CE)], wbuf)
        for rr in range(CHUNK_R):
            for gg in range(8):
                sl = pl.ds(gg * 16, 16)
                r = dstbuf[rr, sl]
                owned = (r >= base_row) & (r < base_row + HALF)
                dstbuf[rr, sl] = jnp.where(owned, r - base_row, 0)
                wsl = pl.ds((rr * 8 + gg) * 16, 16)
                wbuf[wsl] = jnp.where(owned, wbuf[wsl], 0.0)
        hs = [
            pltpu.async_copy(
                agg.at[colbuf.at[j]], rows.at[pl.ds(j * 128, 128)], gsem
            )
            for j in range(CHUNK_R)
        ]
        for h in hs:
            h.wait()
        def mul_body(gi, mc):
            wv = wbuf[pl.ds(gi * 16, 16)]
            for l in range(16):
                w = wv[l]
                e = gi * 16 + l
                for jj in range(4):
                    sl = pl.ds(jj * 16, 16)
                    rows[e, sl] = rows[e, sl] * w
            return mc
        lax.fori_loop(0, CE // 16, mul_body, 0)
        for j in range(CHUNK_R):
            pltpu.sync_copy(
                rows.at[pl.ds(j * 128, 128)], acc.at[dstbuf.at[j]], add=True
            )
        return carry

    lax.fori_loop(0, NCHUNK, chunk_body, 0)

    plsc.subcore_barrier()
    wb = s * STRIPE

    @pl.when(s != NS - 1)
    def _():
        pltpu.sync_copy(
            acc.at[pl.ds(wb, STRIPE)], out.at[pl.ds(base_row + wb, STRIPE)]
        )

    @pl.when(s == NS - 1)
    def _():
        pltpu.sync_copy(
            acc.at[pl.ds(wb, LAST_WB)], out.at[pl.ds(base_row + wb, LAST_WB)]
        )


@functools.cache
def _hop_kernel():
    return pl.kernel(
        _hop_body,
        out_type=jax.ShapeDtypeStruct((N_NODES, DIM), jnp.float32),
        mesh=_mesh(),
        scratch_types=[
            pltpu.VMEM_SHARED((ACC_ROWS, DIM), jnp.float32),
            pltpu.VMEM((CHUNK_R, 128), jnp.int32),
            pltpu.VMEM((CHUNK_R, 128), jnp.int32),
            pltpu.VMEM((CE,), jnp.float32),
            pltpu.VMEM((CE, DIM), jnp.float32),
            pltpu.SemaphoreType.DMA,
        ],
        compiler_params=pltpu.CompilerParams(use_tc_tiling_on_sc=False),
    )


UPT = B // (NC * NS)
NPT = B * K_NEG // (NC * NS)
GROWS = 512


def _gather_body(e1, e2, e3, u2, p2, n2,
                 s1, s2, s3, q1, q2, q3, m1, m2, m3,
                 ubuf, pbuf, nbuf, rows, gsem):
    wid = lax.axis_index("s") * NC + lax.axis_index("c")
    pltpu.sync_copy(u2.at[pl.ds(wid * UPT, UPT)], ubuf)
    pltpu.sync_copy(p2.at[pl.ds(wid * UPT, UPT)], pbuf)
    pltpu.sync_copy(n2.at[pl.ds(wid * NPT, NPT)], nbuf)
    for src, so, po, no in ((e1, s1, q1, m1), (e2, s2, q2, m2), (e3, s3, q3, m3)):
        hu = pltpu.async_copy(src.at[ubuf], rows.at[pl.ds(0, 128)], gsem)
        hp = pltpu.async_copy(src.at[pbuf], rows.at[pl.ds(128, 128)], gsem)
        hu.wait()
        hp.wait()
        pltpu.sync_copy(rows.at[pl.ds(0, 128)], so.at[pl.ds(wid * UPT, UPT)])
        pltpu.sync_copy(rows.at[pl.ds(128, 128)], po.at[pl.ds(wid * UPT, UPT)])
        for q in range(4):
            hs = [
                pltpu.async_copy(
                    src.at[nbuf.at[pl.ds((q * 4 + j) * 128, 128)]],
                    rows.at[pl.ds(j * 128, 128)],
                    gsem,
                )
                for j in range(4)
            ]
            for h in hs:
                h.wait()
            pltpu.sync_copy(
                rows.at[pl.ds(0, GROWS)],
                no.at[pl.ds(wid * NPT + q * GROWS, GROWS)],
            )


_embt = jax.ShapeDtypeStruct((B, DIM), jnp.float32)
_negt = jax.ShapeDtypeStruct((B * K_NEG, DIM), jnp.float32)


@functools.cache
def _gather_kernel():
    return pl.kernel(
        _gather_body,
        out_type=(_embt, _embt, _embt, _embt, _embt, _embt, _negt, _negt, _negt),
        mesh=_mesh(),
        scratch_types=[
            pltpu.VMEM((UPT,), jnp.int32),
            pltpu.VMEM((UPT,), jnp.int32),
            pltpu.VMEM((NPT,), jnp.int32),
            pltpu.VMEM((GROWS, DIM), jnp.float32),
            pltpu.SemaphoreType.DMA,
        ],
        compiler_params=pltpu.CompilerParams(use_tc_tiling_on_sc=False),
    )


BB = 512
GRID = B // BB


def _tail_body(s1, s2, s3, p1, p2, p3, n1, n2, n3, g1, g2, mf_ref, reg_ref):
    i = pl.program_id(0)
    s_pool = (s1[...] + s2[...] + s3[...]) * (1.0 / 3.0)
    pos_e = (p1[...] + p2[...] + p3[...]) * (1.0 / 3.0)
    neg_sum = jnp.zeros((BB, DIM), jnp.float32)
    neg0 = jnp.zeros((BB, DIM), jnp.float32)
    for h, (n, p) in enumerate(((n1, p1), (n2, p2), (n3, p3))):
        nh = n[...].reshape(BB, K_NEG, DIM)
        sc1 = jnp.sum(s_pool[:, None, :] * nh, axis=-1) + g1[h]
        sc2 = jnp.sum(p[...][:, None, :] * nh, axis=-1) + g2[h]
        oh1 = (sc1 >= jnp.max(sc1, axis=1, keepdims=True)).astype(jnp.float32)
        oh2 = (sc2 >= jnp.max(sc2, axis=1, keepdims=True)).astype(jnp.float32)
        ne1 = jnp.sum(oh1[:, :, None] * nh, axis=1)
        ne2 = jnp.sum(oh2[:, :, None] * nh, axis=1)
        negh = (0.5 * ALPHA) * (ne1 + ne2) + (1.0 - ALPHA) * p[...]
        neg_sum = neg_sum + negh
        if h == 0:
            neg0 = negh
    neg_e = neg_sum * (1.0 / 3.0)
    pos_scores = jnp.sum(s_pool * pos_e, axis=-1)
    neg_scores = jnp.sum(s_pool * neg_e, axis=-1)
    mf_part = jnp.sum(jnp.log(1.0 + jnp.exp(neg_scores - pos_scores)))
    reg_part = (
        jnp.sum(s1[...] ** 2) + jnp.sum(p1[...] ** 2) + jnp.sum(neg0 ** 2)
    )

    @pl.when(i == 0)
    def _():
        mf_ref[...] = jnp.zeros((1, 1), jnp.float32)
        reg_ref[...] = jnp.zeros((1, 1), jnp.float32)

    mf_ref[...] = mf_ref[...] + mf_part
    reg_ref[...] = reg_ref[...] + reg_part


_emb_spec = pl.BlockSpec((BB, DIM), lambda i: (i, 0))
_neg_spec = pl.BlockSpec((BB * K_NEG, DIM), lambda i: (i, 0))
_g_spec = pl.BlockSpec((N_HOPS, BB, K_NEG), lambda i: (0, i, 0))
_out_spec = pl.BlockSpec((1, 1), lambda i: (0, 0))

_tail = pl.pallas_call(
    _tail_body,
    grid=(GRID,),
    in_specs=[_emb_spec] * 6 + [_neg_spec] * 3 + [_g_spec] * 2,
    out_specs=(_out_spec, _out_spec),
    out_shape=(
        jax.ShapeDtypeStruct((1, 1), jnp.float32),
        jax.ShapeDtypeStruct((1, 1), jnp.float32),
    ),
)


def kernel(user_embed, item_embed, edge_weight, gumbel1, gumbel2, edge_index,
           users, pos_items, neg_items):
    agg0 = jnp.concatenate([user_embed, item_embed], axis=0)
    pad = EROWS * 128 - NNZ
    rowp = jnp.pad(edge_index[0], (0, pad)).astype(jnp.int32).reshape(EROWS, 128)
    colp = jnp.pad(edge_index[1], (0, pad)).astype(jnp.int32).reshape(EROWS, 128)
    wflat = jnp.pad(edge_weight, (0, pad))
    zeros = jnp.zeros((STRIPE, DIM), jnp.float32)
    hop = _hop_kernel()
    e1 = hop(agg0, rowp, colp, wflat, zeros)
    e2 = hop(e1, rowp, colp, wflat, zeros)
    e3 = hop(e2, rowp, colp, wflat, zeros)
    u2 = users.astype(jnp.int32)
    p2 = (pos_items + N_USERS).astype(jnp.int32)
    n2 = (neg_items.reshape(-1) + N_USERS).astype(jnp.int32)
    s1, s2, s3, q1, q2, q3, m1, m2, m3 = _gather_kernel()(e1, e2, e3, u2, p2, n2)
    g1t = jnp.transpose(gumbel1, (2, 0, 1))
    g2t = jnp.transpose(gumbel2, (2, 0, 1))
    mf_sum, reg_sum = _tail(s1, s2, s3, q1, q2, q3, m1, m2, m3, g1t, g2t)
    mf_loss = mf_sum[0, 0] / B
    emb_loss = DECAY * (reg_sum[0, 0] / 2.0) / B
    return (mf_loss + emb_loss, mf_loss, emb_loss)
